# Initial kernel scaffold; baseline (speedup 1.0000x reference)
import jax
import jax.numpy as jnp
from jax.experimental import pallas as pl


def kernel(x, edge_index, W_in, b_in, g_in, beta_in, tmW0, tmb0, g0, beta0, tmW1, tmb1, g1, beta1, W_out, b_out):
    raise NotImplementedError("write your pallas kernel here")


# speedup vs baseline: 3.6691x; 3.6691x over previous
import functools

import jax
import jax.numpy as jnp
from jax import lax
from jax.experimental import pallas as pl
from jax.experimental.pallas import tpu as pltpu
from jax.experimental.pallas import tpu_sc as plsc

N = 10000
E = 320000
DIN = 128
H = 256
DOUT = 128
GC = 64
REP = 4

NC = 2
NS = 16
CK = 128
CPB = 158
EPAD = NS * CPB * CK
EB = EPAD // 128
TRASH = N
NROW = 10240
RPT = NROW // NS

BN = 1000


def _edge_mask_call(src2, dst2):
    def body(s_ref, d_ref, o_ref):
        s = s_ref[...]
        d = d_ref[...]
        o_ref[...] = jnp.where(s != d, d, TRASH)

    return pl.pallas_call(
        body,
        out_shape=jax.ShapeDtypeStruct((EB, 128), jnp.int32),
    )(src2, dst2)


def _input_call(x, W_in, b_in, g_in, beta_in):
    def body(x_ref, w_ref, b_ref, g_ref, be_ref, o_ref):
        t = jnp.dot(x_ref[...], w_ref[...], preferred_element_type=jnp.float32)
        t = jnp.maximum(t + b_ref[...], 0.0)
        mu = jnp.mean(t, axis=-1, keepdims=True)
        d = t - mu
        var = jnp.mean(d * d, axis=-1, keepdims=True)
        hn = d * lax.rsqrt(var + 1e-5) * g_ref[...] + be_ref[...]
        o_ref[0] = hn[:, :128]
        o_ref[1] = hn[:, 128:]

    return pl.pallas_call(
        body,
        grid=(N // BN,),
        in_specs=[
            pl.BlockSpec((BN, DIN), lambda i: (i, 0)),
            pl.BlockSpec((DIN, H), lambda i: (0, 0)),
            pl.BlockSpec((1, H), lambda i: (0, 0)),
            pl.BlockSpec((1, H), lambda i: (0, 0)),
            pl.BlockSpec((1, H), lambda i: (0, 0)),
        ],
        out_specs=pl.BlockSpec((2, BN, 128), lambda i: (0, i, 0)),
        out_shape=jax.ShapeDtypeStruct((2, N, 128), jnp.float32),
    )(x, W_in, b_in.reshape(1, H), g_in.reshape(1, H), beta_in.reshape(1, H))


def _sc_segsum(h2n, srcp, dstm, z2, z1, with_cnt):
    mesh = plsc.VectorSubcoreMesh(core_axis_name="c", subcore_axis_name="s")

    def body(h_hbm, src_hbm, dst_hbm, z2_hbm, z1_hbm, *rest):
        if with_cnt:
            msum_out, cnt_out = rest[0], rest[1]
            rest = rest[2:]
        else:
            msum_out = rest[0]
            cnt_out = None
            rest = rest[1:]
        src_v, adj_v, dst_v, rows_v, ones_v, msum_sh, cnt_sh, sem = rest

        cid = lax.axis_index("c")
        sid = lax.axis_index("s")
        r0 = sid * RPT

        pltpu.sync_copy(z2_hbm, msum_sh.at[pl.ds(r0, RPT)])
        if with_cnt:
            pltpu.sync_copy(z1_hbm, cnt_sh.at[pl.ds(r0, RPT)])
            for i in range(CK // 16):
                ones_v[pl.ds(i * 16, 16)] = jnp.ones((16,), jnp.float32)
        plsc.subcore_barrier()

        off = cid * N

        def chunk(j, carry):
            eb = (sid * CPB + j) * CK
            pltpu.sync_copy(src_hbm.at[pl.ds(eb, CK)], src_v)
            pltpu.sync_copy(dst_hbm.at[pl.ds(eb, CK)], dst_v)
            for i in range(CK // 16):
                adj_v[pl.ds(i * 16, 16)] = src_v[pl.ds(i * 16, 16)] + off
            pltpu.async_copy(h_hbm.at[adj_v], rows_v, sem).wait()
            pltpu.sync_copy(rows_v, msum_sh.at[dst_v], add=True)
            if with_cnt:
                @pl.when(cid == 0)
                def _():
                    pltpu.sync_copy(ones_v, cnt_sh.at[dst_v], add=True)
            return carry

        lax.fori_loop(0, CPB, chunk, 0)
        plsc.subcore_barrier()

        pltpu.sync_copy(msum_sh.at[pl.ds(r0, RPT)],
                        msum_out.at[cid, pl.ds(r0, RPT)])
        if with_cnt:
            @pl.when(cid == 0)
            def _():
                pltpu.sync_copy(cnt_sh.at[pl.ds(r0, RPT)],
                                cnt_out.at[pl.ds(r0, RPT)])

    if with_cnt:
        out_type = (jax.ShapeDtypeStruct((NC, NROW, 128), jnp.float32),
                    jax.ShapeDtypeStruct((NROW,), jnp.float32))
    else:
        out_type = jax.ShapeDtypeStruct((NC, NROW, 128), jnp.float32)

    fn = pl.kernel(
        body,
        out_type=out_type,
        mesh=mesh,
        scratch_types=[
            pltpu.VMEM((CK,), jnp.int32),
            pltpu.VMEM((CK,), jnp.int32),
            pltpu.VMEM((CK,), jnp.int32),
            pltpu.VMEM((CK, 128), jnp.float32),
            pltpu.VMEM((CK,), jnp.float32),
            pltpu.VMEM_SHARED((NROW, 128), jnp.float32),
            pltpu.VMEM_SHARED((NROW,), jnp.float32),
            pltpu.SemaphoreType.DMA,
        ],
    )
    return fn(h2n, srcp, dstm, z2, z1)


def _gate_call(h_split, msum, cnt2, last_raw, tmW, tmb, g, be,
               W_out=None, b_out=None):
    final = W_out is not None
    has_last = last_raw is not None

    def body(*refs):
        refs = list(refs)
        h_ref = refs.pop(0)
        ms_ref = refs.pop(0)
        cnt_ref = refs.pop(0)
        last_ref = refs.pop(0) if has_last else None
        w_ref = refs.pop(0)
        b_ref = refs.pop(0)
        g_ref = refs.pop(0)
        be_ref = refs.pop(0)
        if final:
            wo_ref = refs.pop(0)
            bo_ref = refs.pop(0)
            o_ref = refs.pop(0)
        else:
            o_ref = refs.pop(0)
            raw_ref = refs.pop(0)

        h0 = h_ref[0]
        h1 = h_ref[1]
        rinv = 1.0 / jnp.maximum(cnt_ref[...], 1.0)
        m0 = ms_ref[0] * rinv
        m1 = ms_ref[1] * rinv

        w = w_ref[...]
        z = (jnp.dot(h0, w[0:128], preferred_element_type=jnp.float32)
             + jnp.dot(h1, w[128:256], preferred_element_type=jnp.float32)
             + jnp.dot(m0, w[256:384], preferred_element_type=jnp.float32)
             + jnp.dot(m1, w[384:512], preferred_element_type=jnp.float32)
             + b_ref[...])
        z = z - jnp.max(z, axis=-1, keepdims=True)
        ez = jnp.exp(z)
        sm = ez / jnp.sum(ez, axis=-1, keepdims=True)

        r_i = lax.broadcasted_iota(jnp.int32, (GC, GC), 0)
        c_i = lax.broadcasted_iota(jnp.int32, (GC, GC), 1)
        tri = (r_i <= c_i).astype(jnp.float32)
        craw = jnp.dot(sm, tri, preferred_element_type=jnp.float32)
        if has_last:
            last = last_ref[...]
            craw = last + (1.0 - last) * craw

        jr = lax.broadcasted_iota(jnp.int32, (GC, 128), 0)
        cc = lax.broadcasted_iota(jnp.int32, (GC, 128), 1)
        rep0 = (jr == cc // REP).astype(jnp.float32)
        rep1 = (jr == GC // 2 + cc // REP).astype(jnp.float32)
        tm0 = jnp.dot(craw, rep0, preferred_element_type=jnp.float32)
        tm1 = jnp.dot(craw, rep1, preferred_element_type=jnp.float32)

        n0 = h0 * tm0 + m0 * (1.0 - tm0)
        n1 = h1 * tm1 + m1 * (1.0 - tm1)

        s = jnp.sum(n0, axis=-1, keepdims=True) + jnp.sum(n1, axis=-1, keepdims=True)
        mu = s / H
        d0 = n0 - mu
        d1 = n1 - mu
        var = (jnp.sum(d0 * d0, axis=-1, keepdims=True)
               + jnp.sum(d1 * d1, axis=-1, keepdims=True)) / H
        rstd = lax.rsqrt(var + 1e-5)
        gv = g_ref[...]
        bev = be_ref[...]
        f0 = d0 * rstd * gv[:, 0:128] + bev[:, 0:128]
        f1 = d1 * rstd * gv[:, 128:256] + bev[:, 128:256]

        if final:
            wo = wo_ref[...]
            o_ref[...] = (jnp.dot(f0, wo[0:128], preferred_element_type=jnp.float32)
                          + jnp.dot(f1, wo[128:256], preferred_element_type=jnp.float32)
                          + bo_ref[...])
        else:
            o_ref[0] = f0
            o_ref[1] = f1
            raw_ref[...] = craw

    in_specs = [
        pl.BlockSpec((2, BN, 128), lambda i: (0, i, 0)),
        pl.BlockSpec((2, BN, 128), lambda i: (0, i, 0)),
        pl.BlockSpec((BN, 1), lambda i: (i, 0)),
    ]
    args = [h_split, msum, cnt2]
    if has_last:
        in_specs.append(pl.BlockSpec((BN, GC), lambda i: (i, 0)))
        args.append(last_raw)
    in_specs += [
        pl.BlockSpec((2 * H, GC), lambda i: (0, 0)),
        pl.BlockSpec((1, GC), lambda i: (0, 0)),
        pl.BlockSpec((1, H), lambda i: (0, 0)),
        pl.BlockSpec((1, H), lambda i: (0, 0)),
    ]
    args += [tmW, tmb.reshape(1, GC), g.reshape(1, H), be.reshape(1, H)]
    if final:
        in_specs += [
            pl.BlockSpec((H, DOUT), lambda i: (0, 0)),
            pl.BlockSpec((1, DOUT), lambda i: (0, 0)),
        ]
        args += [W_out, b_out.reshape(1, DOUT)]
        out_specs = pl.BlockSpec((BN, DOUT), lambda i: (i, 0))
        out_shape = jax.ShapeDtypeStruct((N, DOUT), jnp.float32)
    else:
        out_specs = (
            pl.BlockSpec((2, BN, 128), lambda i: (0, i, 0)),
            pl.BlockSpec((BN, GC), lambda i: (i, 0)),
        )
        out_shape = (
            jax.ShapeDtypeStruct((2, N, 128), jnp.float32),
            jax.ShapeDtypeStruct((N, GC), jnp.float32),
        )

    return pl.pallas_call(
        body,
        grid=(N // BN,),
        in_specs=in_specs,
        out_specs=out_specs,
        out_shape=out_shape,
    )(*args)


@jax.jit
def kernel(x, edge_index, W_in, b_in, g_in, beta_in,
           tmW0, tmb0, g0, beta0, tmW1, tmb1, g1, beta1, W_out, b_out):
    src = edge_index[0]
    dst = edge_index[1]
    pad = jnp.zeros((EPAD - E,), jnp.int32)
    srcp = jnp.concatenate([src, pad])
    dstp = jnp.concatenate([dst, pad])

    dstm = _edge_mask_call(srcp.reshape(EB, 128),
                           dstp.reshape(EB, 128)).reshape(EPAD)

    h_split = _input_call(x, W_in, b_in, g_in, beta_in)

    z2 = jnp.zeros((RPT, 128), jnp.float32)
    z1 = jnp.zeros((RPT,), jnp.float32)

    msum1, cnt = _sc_segsum(h_split.reshape(2 * N, 128), srcp, dstm,
                            z2, z1, with_cnt=True)
    cnt2 = cnt.reshape(NROW, 1)

    h_split2, raw1 = _gate_call(h_split, msum1, cnt2, None,
                                tmW0, tmb0, g0, beta0)

    msum2 = _sc_segsum(h_split2.reshape(2 * N, 128), srcp, dstm,
                       z2, z1, with_cnt=False)

    return _gate_call(h_split2, msum2, cnt2, raw1,
                      tmW1, tmb1, g1, beta1, W_out=W_out, b_out=b_out)


# speedup vs baseline: 4.0138x; 1.0939x over previous
import functools

import jax
import jax.numpy as jnp
from jax import lax
from jax.experimental import pallas as pl
from jax.experimental.pallas import tpu as pltpu
from jax.experimental.pallas import tpu_sc as plsc

N = 10000
E = 320000
DIN = 128
H = 256
DOUT = 128
GC = 64
REP = 4

NC = 2
NS = 16
CK = 128
NBUF = 2
CPB = 160
EPAD = NS * CPB * CK
EB = EPAD // 128
TRASH = N
NROW = 10240
RPT = NROW // NS

BN = 1000


def _edge_mask_call(src2, dst2):
    def body(s_ref, d_ref, o_ref, sa_ref):
        s = s_ref[...]
        d = d_ref[...]
        o_ref[...] = jnp.where(s != d, d, TRASH)
        sa_ref[0] = s
        sa_ref[1] = s + N

    return pl.pallas_call(
        body,
        out_shape=(jax.ShapeDtypeStruct((EB, 128), jnp.int32),
                   jax.ShapeDtypeStruct((2, EB, 128), jnp.int32)),
    )(src2, dst2)


def _input_call(x, W_in, b_in, g_in, beta_in):
    def body(x_ref, w_ref, b_ref, g_ref, be_ref, o_ref):
        t = jnp.dot(x_ref[...], w_ref[...], preferred_element_type=jnp.float32)
        t = jnp.maximum(t + b_ref[...], 0.0)
        mu = jnp.mean(t, axis=-1, keepdims=True)
        d = t - mu
        var = jnp.mean(d * d, axis=-1, keepdims=True)
        hn = d * lax.rsqrt(var + 1e-5) * g_ref[...] + be_ref[...]
        o_ref[0] = hn[:, :128]
        o_ref[1] = hn[:, 128:]

    return pl.pallas_call(
        body,
        grid=(N // BN,),
        in_specs=[
            pl.BlockSpec((BN, DIN), lambda i: (i, 0)),
            pl.BlockSpec((DIN, H), lambda i: (0, 0)),
            pl.BlockSpec((1, H), lambda i: (0, 0)),
            pl.BlockSpec((1, H), lambda i: (0, 0)),
            pl.BlockSpec((1, H), lambda i: (0, 0)),
        ],
        out_specs=pl.BlockSpec((2, BN, 128), lambda i: (0, i, 0)),
        out_shape=jax.ShapeDtypeStruct((2, N, 128), jnp.float32),
    )(x, W_in, b_in.reshape(1, H), g_in.reshape(1, H), beta_in.reshape(1, H))


def _sc_segsum(h2n, srcadj, dstm2, z2, z1, with_cnt):
    mesh = plsc.VectorSubcoreMesh(core_axis_name="c", subcore_axis_name="s")

    def body(h_hbm, src_hbm, dst_hbm, z2_hbm, z1_hbm, *rest):
        if with_cnt:
            msum_out, cnt_out = rest[0], rest[1]
            rest = rest[2:]
        else:
            msum_out = rest[0]
            cnt_out = None
            rest = rest[1:]
        ones_v, msum_sh, cnt_sh = rest[:3]
        rest = rest[3:]
        srcs = rest[0:NBUF]
        dsts = rest[NBUF:2 * NBUF]
        rows = rest[2 * NBUF:3 * NBUF]
        gsem = rest[3 * NBUF:4 * NBUF]
        isem = rest[4 * NBUF:5 * NBUF]

        cid = lax.axis_index("c")
        sid = lax.axis_index("s")
        r0 = sid * RPT
        e0 = sid * CPB

        pltpu.sync_copy(z2_hbm, msum_sh.at[pl.ds(r0, RPT)])
        if with_cnt:
            pltpu.sync_copy(z1_hbm, cnt_sh.at[pl.ds(r0, RPT)])
            for i in range(CK // 16):
                ones_v[pl.ds(i * 16, 16)] = jnp.ones((16,), jnp.float32)
        plsc.subcore_barrier()

        for b in range(NBUF):
            pltpu.async_copy(src_hbm.at[cid, e0 + b], srcs[b], isem[b])
            pltpu.async_copy(dst_hbm.at[e0 + b], dsts[b], isem[b])
        pltpu.make_async_copy(src_hbm.at[cid, e0], srcs[0], isem[0]).wait()
        pltpu.make_async_copy(dst_hbm.at[e0], dsts[0], isem[0]).wait()
        pltpu.async_copy(h_hbm.at[srcs[0]], rows[0], gsem[0])

        def outer(jo, carry):
            for b in range(NBUF):
                j = jo * NBUF + b
                nb = (b + 1) % NBUF
                pltpu.make_async_copy(h_hbm.at[srcs[b]], rows[b],
                                      gsem[b]).wait()

                @pl.when(j + 1 < CPB)
                def _():
                    pltpu.make_async_copy(src_hbm.at[cid, e0 + j + 1],
                                          srcs[nb], isem[nb]).wait()
                    pltpu.make_async_copy(dst_hbm.at[e0 + j + 1],
                                          dsts[nb], isem[nb]).wait()
                    pltpu.async_copy(h_hbm.at[srcs[nb]], rows[nb], gsem[nb])

                pltpu.sync_copy(rows[b], msum_sh.at[dsts[b]], add=True)
                if with_cnt:
                    @pl.when(cid == 0)
                    def _():
                        pltpu.sync_copy(ones_v, cnt_sh.at[dsts[b]], add=True)

                @pl.when(j + NBUF < CPB)
                def _():
                    pltpu.async_copy(src_hbm.at[cid, e0 + j + NBUF],
                                     srcs[b], isem[b])
                    pltpu.async_copy(dst_hbm.at[e0 + j + NBUF],
                                     dsts[b], isem[b])
            return carry

        lax.fori_loop(0, CPB // NBUF, outer, 0)
        plsc.subcore_barrier()

        pltpu.sync_copy(msum_sh.at[pl.ds(r0, RPT)],
                        msum_out.at[cid, pl.ds(r0, RPT)])
        if with_cnt:
            @pl.when(cid == 0)
            def _():
                pltpu.sync_copy(cnt_sh.at[pl.ds(r0, RPT)],
                                cnt_out.at[pl.ds(r0, RPT)])

    if with_cnt:
        out_type = (jax.ShapeDtypeStruct((NC, NROW, 128), jnp.float32),
                    jax.ShapeDtypeStruct((NROW,), jnp.float32))
    else:
        out_type = jax.ShapeDtypeStruct((NC, NROW, 128), jnp.float32)

    fn = pl.kernel(
        body,
        out_type=out_type,
        mesh=mesh,
        scratch_types=(
            [
                pltpu.VMEM((CK,), jnp.float32),
                pltpu.VMEM_SHARED((NROW, 128), jnp.float32),
                pltpu.VMEM_SHARED((NROW,), jnp.float32),
            ]
            + [pltpu.VMEM((CK,), jnp.int32) for _ in range(NBUF)]
            + [pltpu.VMEM((CK,), jnp.int32) for _ in range(NBUF)]
            + [pltpu.VMEM((CK, 128), jnp.float32) for _ in range(NBUF)]
            + [pltpu.SemaphoreType.DMA for _ in range(NBUF)]
            + [pltpu.SemaphoreType.DMA for _ in range(NBUF)]
        ),
    )
    return fn(h2n, srcadj, dstm2, z2, z1)


def _gate_call(h_split, msum, cnt2, last_raw, tmW, tmb, g, be,
               W_out=None, b_out=None):
    final = W_out is not None
    has_last = last_raw is not None

    def body(*refs):
        refs = list(refs)
        h_ref = refs.pop(0)
        ms_ref = refs.pop(0)
        cnt_ref = refs.pop(0)
        last_ref = refs.pop(0) if has_last else None
        w_ref = refs.pop(0)
        b_ref = refs.pop(0)
        g_ref = refs.pop(0)
        be_ref = refs.pop(0)
        if final:
            wo_ref = refs.pop(0)
            bo_ref = refs.pop(0)
            o_ref = refs.pop(0)
        else:
            o_ref = refs.pop(0)
            raw_ref = refs.pop(0)

        h0 = h_ref[0]
        h1 = h_ref[1]
        rinv = 1.0 / jnp.maximum(cnt_ref[...], 1.0)
        m0 = ms_ref[0] * rinv
        m1 = ms_ref[1] * rinv

        w = w_ref[...]
        z = (jnp.dot(h0, w[0:128], preferred_element_type=jnp.float32)
             + jnp.dot(h1, w[128:256], preferred_element_type=jnp.float32)
             + jnp.dot(m0, w[256:384], preferred_element_type=jnp.float32)
             + jnp.dot(m1, w[384:512], preferred_element_type=jnp.float32)
             + b_ref[...])
        z = z - jnp.max(z, axis=-1, keepdims=True)
        ez = jnp.exp(z)
        sm = ez / jnp.sum(ez, axis=-1, keepdims=True)

        r_i = lax.broadcasted_iota(jnp.int32, (GC, GC), 0)
        c_i = lax.broadcasted_iota(jnp.int32, (GC, GC), 1)
        tri = (r_i <= c_i).astype(jnp.float32)
        craw = jnp.dot(sm, tri, preferred_element_type=jnp.float32)
        if has_last:
            last = last_ref[...]
            craw = last + (1.0 - last) * craw

        jr = lax.broadcasted_iota(jnp.int32, (GC, 128), 0)
        cc = lax.broadcasted_iota(jnp.int32, (GC, 128), 1)
        rep0 = (jr == cc // REP).astype(jnp.float32)
        rep1 = (jr == GC // 2 + cc // REP).astype(jnp.float32)
        tm0 = jnp.dot(craw, rep0, preferred_element_type=jnp.float32)
        tm1 = jnp.dot(craw, rep1, preferred_element_type=jnp.float32)

        n0 = h0 * tm0 + m0 * (1.0 - tm0)
        n1 = h1 * tm1 + m1 * (1.0 - tm1)

        s = jnp.sum(n0, axis=-1, keepdims=True) + jnp.sum(n1, axis=-1, keepdims=True)
        mu = s / H
        d0 = n0 - mu
        d1 = n1 - mu
        var = (jnp.sum(d0 * d0, axis=-1, keepdims=True)
               + jnp.sum(d1 * d1, axis=-1, keepdims=True)) / H
        rstd = lax.rsqrt(var + 1e-5)
        gv = g_ref[...]
        bev = be_ref[...]
        f0 = d0 * rstd * gv[:, 0:128] + bev[:, 0:128]
        f1 = d1 * rstd * gv[:, 128:256] + bev[:, 128:256]

        if final:
            wo = wo_ref[...]
            o_ref[...] = (jnp.dot(f0, wo[0:128], preferred_element_type=jnp.float32)
                          + jnp.dot(f1, wo[128:256], preferred_element_type=jnp.float32)
                          + bo_ref[...])
        else:
            o_ref[0] = f0
            o_ref[1] = f1
            raw_ref[...] = craw

    in_specs = [
        pl.BlockSpec((2, BN, 128), lambda i: (0, i, 0)),
        pl.BlockSpec((2, BN, 128), lambda i: (0, i, 0)),
        pl.BlockSpec((BN, 1), lambda i: (i, 0)),
    ]
    args = [h_split, msum, cnt2]
    if has_last:
        in_specs.append(pl.BlockSpec((BN, GC), lambda i: (i, 0)))
        args.append(last_raw)
    in_specs += [
        pl.BlockSpec((2 * H, GC), lambda i: (0, 0)),
        pl.BlockSpec((1, GC), lambda i: (0, 0)),
        pl.BlockSpec((1, H), lambda i: (0, 0)),
        pl.BlockSpec((1, H), lambda i: (0, 0)),
    ]
    args += [tmW, tmb.reshape(1, GC), g.reshape(1, H), be.reshape(1, H)]
    if final:
        in_specs += [
            pl.BlockSpec((H, DOUT), lambda i: (0, 0)),
            pl.BlockSpec((1, DOUT), lambda i: (0, 0)),
        ]
        args += [W_out, b_out.reshape(1, DOUT)]
        out_specs = pl.BlockSpec((BN, DOUT), lambda i: (i, 0))
        out_shape = jax.ShapeDtypeStruct((N, DOUT), jnp.float32)
    else:
        out_specs = (
            pl.BlockSpec((2, BN, 128), lambda i: (0, i, 0)),
            pl.BlockSpec((BN, GC), lambda i: (i, 0)),
        )
        out_shape = (
            jax.ShapeDtypeStruct((2, N, 128), jnp.float32),
            jax.ShapeDtypeStruct((N, GC), jnp.float32),
        )

    return pl.pallas_call(
        body,
        grid=(N // BN,),
        in_specs=in_specs,
        out_specs=out_specs,
        out_shape=out_shape,
    )(*args)


@jax.jit
def kernel(x, edge_index, W_in, b_in, g_in, beta_in,
           tmW0, tmb0, g0, beta0, tmW1, tmb1, g1, beta1, W_out, b_out):
    src = edge_index[0]
    dst = edge_index[1]
    pad = jnp.zeros((EPAD - E,), jnp.int32)
    srcp = jnp.concatenate([src, pad])
    dstp = jnp.concatenate([dst, pad])

    dstm2, srcadj = _edge_mask_call(srcp.reshape(EB, 128),
                                    dstp.reshape(EB, 128))

    h_split = _input_call(x, W_in, b_in, g_in, beta_in)

    z2 = jnp.zeros((RPT, 128), jnp.float32)
    z1 = jnp.zeros((RPT,), jnp.float32)

    msum1, cnt = _sc_segsum(h_split.reshape(2 * N, 128), srcadj, dstm2,
                            z2, z1, with_cnt=True)
    cnt2 = cnt.reshape(NROW, 1)

    h_split2, raw1 = _gate_call(h_split, msum1, cnt2, None,
                                tmW0, tmb0, g0, beta0)

    msum2 = _sc_segsum(h_split2.reshape(2 * N, 128), srcadj, dstm2,
                       z2, z1, with_cnt=False)

    return _gate_call(h_split2, msum2, cnt2, raw1,
                      tmW1, tmb1, g1, beta1, W_out=W_out, b_out=b_out)


# speedup vs baseline: 4.1432x; 1.0322x over previous
import functools

import jax
import jax.numpy as jnp
from jax import lax
from jax.experimental import pallas as pl
from jax.experimental.pallas import tpu as pltpu
from jax.experimental.pallas import tpu_sc as plsc

N = 10000
E = 320000
DIN = 128
H = 256
DOUT = 128
GC = 64
REP = 4

NC = 2
NS = 16
CK = 64
NBUF = 4
NIDX = 8
CPB = 320
EPAD = NS * CPB * CK
EB = EPAD // 128
ECH = EPAD // CK
TRASH = N
NROW = 10240
RPT = NROW // NS

BN = 1000


def _edge_mask_call(src2, dst2):
    def body(s_ref, d_ref, o_ref, sa_ref):
        s = s_ref[...]
        d = d_ref[...]
        o_ref[...] = jnp.where(s != d, d, TRASH)
        sa_ref[0] = s
        sa_ref[1] = s + N

    return pl.pallas_call(
        body,
        out_shape=(jax.ShapeDtypeStruct((EB, 128), jnp.int32),
                   jax.ShapeDtypeStruct((2, EB, 128), jnp.int32)),
    )(src2, dst2)


def _input_call(x, W_in, b_in, g_in, beta_in):
    def body(x_ref, w_ref, b_ref, g_ref, be_ref, o_ref):
        t = jnp.dot(x_ref[...], w_ref[...], preferred_element_type=jnp.float32)
        t = jnp.maximum(t + b_ref[...], 0.0)
        mu = jnp.mean(t, axis=-1, keepdims=True)
        d = t - mu
        var = jnp.mean(d * d, axis=-1, keepdims=True)
        hn = d * lax.rsqrt(var + 1e-5) * g_ref[...] + be_ref[...]
        o_ref[0] = hn[:, :128]
        o_ref[1] = hn[:, 128:]

    return pl.pallas_call(
        body,
        grid=(N // BN,),
        in_specs=[
            pl.BlockSpec((BN, DIN), lambda i: (i, 0)),
            pl.BlockSpec((DIN, H), lambda i: (0, 0)),
            pl.BlockSpec((1, H), lambda i: (0, 0)),
            pl.BlockSpec((1, H), lambda i: (0, 0)),
            pl.BlockSpec((1, H), lambda i: (0, 0)),
        ],
        out_specs=pl.BlockSpec((2, BN, 128), lambda i: (0, i, 0)),
        out_shape=jax.ShapeDtypeStruct((2, N, 128), jnp.float32),
    )(x, W_in, b_in.reshape(1, H), g_in.reshape(1, H), beta_in.reshape(1, H))


def _sc_segsum(h2n, srcadj, dstm2, z2, z1, with_cnt):
    mesh = plsc.VectorSubcoreMesh(core_axis_name="c", subcore_axis_name="s")

    def body(h_hbm, src_hbm, dst_hbm, z2_hbm, z1_hbm, *rest):
        if with_cnt:
            msum_out, cnt_out = rest[0], rest[1]
            rest = rest[2:]
        else:
            msum_out = rest[0]
            cnt_out = None
            rest = rest[1:]
        ones_v, msum_sh, cnt_sh = rest[:3]
        rest = rest[3:]
        srcs = rest[0:NIDX]
        dsts = rest[NIDX:2 * NIDX]
        rows = rest[2 * NIDX:2 * NIDX + NBUF]
        rest = rest[2 * NIDX + NBUF:]
        gsem = rest[0:NBUF]
        ssem = rest[NBUF:2 * NBUF]
        isem = rest[2 * NBUF:2 * NBUF + NIDX]

        cid = lax.axis_index("c")
        sid = lax.axis_index("s")
        r0 = sid * RPT
        e0 = sid * CPB

        pltpu.sync_copy(z2_hbm, msum_sh.at[pl.ds(r0, RPT)])
        if with_cnt:
            pltpu.sync_copy(z1_hbm, cnt_sh.at[pl.ds(r0, RPT)])
            for i in range(CK // 16):
                ones_v[pl.ds(i * 16, 16)] = jnp.ones((16,), jnp.float32)
        plsc.subcore_barrier()

        def idx_wait(q, j):
            pltpu.make_async_copy(src_hbm.at[cid, e0 + j], srcs[q],
                                  isem[q]).wait()
            pltpu.make_async_copy(dst_hbm.at[e0 + j], dsts[q],
                                  isem[q]).wait()

        for q in range(NIDX):
            pltpu.async_copy(src_hbm.at[cid, e0 + q], srcs[q], isem[q])
            pltpu.async_copy(dst_hbm.at[e0 + q], dsts[q], isem[q])
        for j in range(2):
            idx_wait(j, j)
            pltpu.async_copy(h_hbm.at[srcs[j]], rows[j], gsem[j])

        def outer(jo, carry):
            for u in range(NIDX):
                j = jo * NIDX + u
                b = u % NBUF
                q = u
                pltpu.make_async_copy(h_hbm.at[srcs[q]], rows[b],
                                      gsem[b]).wait()
                pltpu.async_copy(rows[b], msum_sh.at[dsts[q]], ssem[b],
                                 add=True)
                if with_cnt:
                    @pl.when(cid == 0)
                    def _():
                        pltpu.sync_copy(ones_v, cnt_sh.at[dsts[q]], add=True)

                b2 = (u + NBUF - 2) % NBUF
                q2 = (u + NIDX - 2) % NIDX

                @pl.when(j >= 2)
                def _():
                    pltpu.make_async_copy(rows[b2],
                                          msum_sh.at[dsts[q2]],
                                          ssem[b2]).wait()

                @pl.when((j >= 2) & (j + NIDX - 2 < CPB))
                def _():
                    pltpu.async_copy(src_hbm.at[cid, e0 + j + NIDX - 2],
                                     srcs[q2], isem[q2])
                    pltpu.async_copy(dst_hbm.at[e0 + j + NIDX - 2],
                                     dsts[q2], isem[q2])

                bq = (u + 2) % NBUF
                qq = (u + 2) % NIDX

                @pl.when(j + 2 < CPB)
                def _():
                    idx_wait(qq, j + 2)
                    pltpu.async_copy(h_hbm.at[srcs[qq]], rows[bq], gsem[bq])
            return carry

        lax.fori_loop(0, CPB // NIDX, outer, 0)
        pltpu.make_async_copy(rows[(CPB - 2) % NBUF],
                              msum_sh.at[dsts[(CPB - 2) % NIDX]],
                              ssem[(CPB - 2) % NBUF]).wait()
        pltpu.make_async_copy(rows[(CPB - 1) % NBUF],
                              msum_sh.at[dsts[(CPB - 1) % NIDX]],
                              ssem[(CPB - 1) % NBUF]).wait()
        plsc.subcore_barrier()

        pltpu.sync_copy(msum_sh.at[pl.ds(r0, RPT)],
                        msum_out.at[cid, pl.ds(r0, RPT)])
        if with_cnt:
            @pl.when(cid == 0)
            def _():
                pltpu.sync_copy(cnt_sh.at[pl.ds(r0, RPT)],
                                cnt_out.at[pl.ds(r0, RPT)])

    if with_cnt:
        out_type = (jax.ShapeDtypeStruct((NC, NROW, 128), jnp.float32),
                    jax.ShapeDtypeStruct((NROW,), jnp.float32))
    else:
        out_type = jax.ShapeDtypeStruct((NC, NROW, 128), jnp.float32)

    fn = pl.kernel(
        body,
        out_type=out_type,
        mesh=mesh,
        scratch_types=(
            [
                pltpu.VMEM((CK,), jnp.float32),
                pltpu.VMEM_SHARED((NROW, 128), jnp.float32),
                pltpu.VMEM_SHARED((NROW,), jnp.float32),
            ]
            + [pltpu.VMEM((CK,), jnp.int32) for _ in range(NIDX)]
            + [pltpu.VMEM((CK,), jnp.int32) for _ in range(NIDX)]
            + [pltpu.VMEM((CK, 128), jnp.float32) for _ in range(NBUF)]
            + [pltpu.SemaphoreType.DMA for _ in range(NBUF)]
            + [pltpu.SemaphoreType.DMA for _ in range(NBUF)]
            + [pltpu.SemaphoreType.DMA for _ in range(NIDX)]
        ),
    )
    return fn(h2n, srcadj, dstm2, z2, z1)


def _gate_call(h_split, msum, cnt2, last_raw, tmW, tmb, g, be,
               W_out=None, b_out=None):
    final = W_out is not None
    has_last = last_raw is not None

    def body(*refs):
        refs = list(refs)
        h_ref = refs.pop(0)
        ms_ref = refs.pop(0)
        cnt_ref = refs.pop(0)
        last_ref = refs.pop(0) if has_last else None
        w_ref = refs.pop(0)
        b_ref = refs.pop(0)
        g_ref = refs.pop(0)
        be_ref = refs.pop(0)
        if final:
            wo_ref = refs.pop(0)
            bo_ref = refs.pop(0)
            o_ref = refs.pop(0)
        else:
            o_ref = refs.pop(0)
            raw_ref = refs.pop(0)

        h0 = h_ref[0]
        h1 = h_ref[1]
        rinv = 1.0 / jnp.maximum(cnt_ref[...], 1.0)
        m0 = ms_ref[0] * rinv
        m1 = ms_ref[1] * rinv

        w = w_ref[...]
        z = (jnp.dot(h0, w[0:128], preferred_element_type=jnp.float32)
             + jnp.dot(h1, w[128:256], preferred_element_type=jnp.float32)
             + jnp.dot(m0, w[256:384], preferred_element_type=jnp.float32)
             + jnp.dot(m1, w[384:512], preferred_element_type=jnp.float32)
             + b_ref[...])
        z = z - jnp.max(z, axis=-1, keepdims=True)
        ez = jnp.exp(z)
        sm = ez / jnp.sum(ez, axis=-1, keepdims=True)

        r_i = lax.broadcasted_iota(jnp.int32, (GC, GC), 0)
        c_i = lax.broadcasted_iota(jnp.int32, (GC, GC), 1)
        tri = (r_i <= c_i).astype(jnp.float32)
        craw = jnp.dot(sm, tri, preferred_element_type=jnp.float32)
        if has_last:
            last = last_ref[...]
            craw = last + (1.0 - last) * craw

        jr = lax.broadcasted_iota(jnp.int32, (GC, 128), 0)
        cc = lax.broadcasted_iota(jnp.int32, (GC, 128), 1)
        rep0 = (jr == cc // REP).astype(jnp.float32)
        rep1 = (jr == GC // 2 + cc // REP).astype(jnp.float32)
        tm0 = jnp.dot(craw, rep0, preferred_element_type=jnp.float32)
        tm1 = jnp.dot(craw, rep1, preferred_element_type=jnp.float32)

        n0 = h0 * tm0 + m0 * (1.0 - tm0)
        n1 = h1 * tm1 + m1 * (1.0 - tm1)

        s = jnp.sum(n0, axis=-1, keepdims=True) + jnp.sum(n1, axis=-1, keepdims=True)
        mu = s / H
        d0 = n0 - mu
        d1 = n1 - mu
        var = (jnp.sum(d0 * d0, axis=-1, keepdims=True)
               + jnp.sum(d1 * d1, axis=-1, keepdims=True)) / H
        rstd = lax.rsqrt(var + 1e-5)
        gv = g_ref[...]
        bev = be_ref[...]
        f0 = d0 * rstd * gv[:, 0:128] + bev[:, 0:128]
        f1 = d1 * rstd * gv[:, 128:256] + bev[:, 128:256]

        if final:
            wo = wo_ref[...]
            o_ref[...] = (jnp.dot(f0, wo[0:128], preferred_element_type=jnp.float32)
                          + jnp.dot(f1, wo[128:256], preferred_element_type=jnp.float32)
                          + bo_ref[...])
        else:
            o_ref[0] = f0
            o_ref[1] = f1
            raw_ref[...] = craw

    in_specs = [
        pl.BlockSpec((2, BN, 128), lambda i: (0, i, 0)),
        pl.BlockSpec((2, BN, 128), lambda i: (0, i, 0)),
        pl.BlockSpec((BN, 1), lambda i: (i, 0)),
    ]
    args = [h_split, msum, cnt2]
    if has_last:
        in_specs.append(pl.BlockSpec((BN, GC), lambda i: (i, 0)))
        args.append(last_raw)
    in_specs += [
        pl.BlockSpec((2 * H, GC), lambda i: (0, 0)),
        pl.BlockSpec((1, GC), lambda i: (0, 0)),
        pl.BlockSpec((1, H), lambda i: (0, 0)),
        pl.BlockSpec((1, H), lambda i: (0, 0)),
    ]
    args += [tmW, tmb.reshape(1, GC), g.reshape(1, H), be.reshape(1, H)]
    if final:
        in_specs += [
            pl.BlockSpec((H, DOUT), lambda i: (0, 0)),
            pl.BlockSpec((1, DOUT), lambda i: (0, 0)),
        ]
        args += [W_out, b_out.reshape(1, DOUT)]
        out_specs = pl.BlockSpec((BN, DOUT), lambda i: (i, 0))
        out_shape = jax.ShapeDtypeStruct((N, DOUT), jnp.float32)
    else:
        out_specs = (
            pl.BlockSpec((2, BN, 128), lambda i: (0, i, 0)),
            pl.BlockSpec((BN, GC), lambda i: (i, 0)),
        )
        out_shape = (
            jax.ShapeDtypeStruct((2, N, 128), jnp.float32),
            jax.ShapeDtypeStruct((N, GC), jnp.float32),
        )

    return pl.pallas_call(
        body,
        grid=(N // BN,),
        in_specs=in_specs,
        out_specs=out_specs,
        out_shape=out_shape,
    )(*args)


@jax.jit
def kernel(x, edge_index, W_in, b_in, g_in, beta_in,
           tmW0, tmb0, g0, beta0, tmW1, tmb1, g1, beta1, W_out, b_out):
    src = edge_index[0]
    dst = edge_index[1]
    pad = jnp.zeros((EPAD - E,), jnp.int32)
    srcp = jnp.concatenate([src, pad])
    dstp = jnp.concatenate([dst, pad])

    dstm2, srcadj = _edge_mask_call(srcp.reshape(EB, 128),
                                    dstp.reshape(EB, 128))

    h_split = _input_call(x, W_in, b_in, g_in, beta_in)

    z2 = jnp.zeros((RPT, 128), jnp.float32)
    z1 = jnp.zeros((RPT,), jnp.float32)

    srcc = srcadj.reshape(2, ECH, CK)
    dstc = dstm2.reshape(ECH, CK)

    msum1, cnt = _sc_segsum(h_split.reshape(2 * N, 128), srcc, dstc,
                            z2, z1, with_cnt=True)
    cnt2 = cnt.reshape(NROW, 1)

    h_split2, raw1 = _gate_call(h_split, msum1, cnt2, None,
                                tmW0, tmb0, g0, beta0)

    msum2 = _sc_segsum(h_split2.reshape(2 * N, 128), srcc, dstc,
                       z2, z1, with_cnt=False)

    return _gate_call(h_split2, msum2, cnt2, raw1,
                      tmW1, tmb1, g1, beta1, W_out=W_out, b_out=b_out)


# speedup vs baseline: 4.2555x; 1.0271x over previous
import functools
from math import gcd as _gcd

import jax
import jax.numpy as jnp
from jax import lax
from jax.experimental import pallas as pl
from jax.experimental.pallas import tpu as pltpu
from jax.experimental.pallas import tpu_sc as plsc

N = 10000
E = 320000
DIN = 128
H = 256
DOUT = 128
GC = 64
REP = 4

NC = 2
NS = 16
CK = 64
NBUF = 5
GA = NBUF - 2
NIDX = 10
CPB = 320
EPAD = NS * CPB * CK
EB = EPAD // 128
ECH = EPAD // CK
TRASH = N
NROW = 10240
RPT = NROW // NS

BN = 1000


def _edge_mask_call(src2, dst2):
    def body(s_ref, d_ref, o_ref, sa_ref):
        s = s_ref[...]
        d = d_ref[...]
        o_ref[...] = jnp.where(s != d, d, TRASH)
        sa_ref[0] = s
        sa_ref[1] = s + N

    return pl.pallas_call(
        body,
        out_shape=(jax.ShapeDtypeStruct((EB, 128), jnp.int32),
                   jax.ShapeDtypeStruct((2, EB, 128), jnp.int32)),
    )(src2, dst2)


def _input_call(x, W_in, b_in, g_in, beta_in):
    def body(x_ref, w_ref, b_ref, g_ref, be_ref, o_ref):
        t = jnp.dot(x_ref[...], w_ref[...], preferred_element_type=jnp.float32)
        t = jnp.maximum(t + b_ref[...], 0.0)
        mu = jnp.mean(t, axis=-1, keepdims=True)
        d = t - mu
        var = jnp.mean(d * d, axis=-1, keepdims=True)
        hn = d * lax.rsqrt(var + 1e-5) * g_ref[...] + be_ref[...]
        o_ref[0] = hn[:, :128]
        o_ref[1] = hn[:, 128:]

    return pl.pallas_call(
        body,
        grid=(N // BN,),
        in_specs=[
            pl.BlockSpec((BN, DIN), lambda i: (i, 0)),
            pl.BlockSpec((DIN, H), lambda i: (0, 0)),
            pl.BlockSpec((1, H), lambda i: (0, 0)),
            pl.BlockSpec((1, H), lambda i: (0, 0)),
            pl.BlockSpec((1, H), lambda i: (0, 0)),
        ],
        out_specs=pl.BlockSpec((2, BN, 128), lambda i: (0, i, 0)),
        out_shape=jax.ShapeDtypeStruct((2, N, 128), jnp.float32),
    )(x, W_in, b_in.reshape(1, H), g_in.reshape(1, H), beta_in.reshape(1, H))


def _sc_segsum(h2n, srcadj, dstm2, z2, z1, with_cnt):
    mesh = plsc.VectorSubcoreMesh(core_axis_name="c", subcore_axis_name="s")

    def body(h_hbm, src_hbm, dst_hbm, z2_hbm, z1_hbm, *rest):
        if with_cnt:
            msum_out, cnt_out = rest[0], rest[1]
            rest = rest[2:]
        else:
            msum_out = rest[0]
            cnt_out = None
            rest = rest[1:]
        ones_v, msum_sh, cnt_sh = rest[:3]
        rest = rest[3:]
        srcs = rest[0:NIDX]
        dsts = rest[NIDX:2 * NIDX]
        rows = rest[2 * NIDX:2 * NIDX + NBUF]
        rest = rest[2 * NIDX + NBUF:]
        gsem = rest[0:NBUF]
        ssem = rest[NBUF:2 * NBUF]
        isem = rest[2 * NBUF:2 * NBUF + NIDX]

        cid = lax.axis_index("c")
        sid = lax.axis_index("s")
        r0 = sid * RPT
        e0 = sid * CPB

        pltpu.sync_copy(z2_hbm, msum_sh.at[pl.ds(r0, RPT)])
        if with_cnt:
            pltpu.sync_copy(z1_hbm, cnt_sh.at[pl.ds(r0, RPT)])
            for i in range(CK // 16):
                ones_v[pl.ds(i * 16, 16)] = jnp.ones((16,), jnp.float32)
        plsc.subcore_barrier()

        def idx_wait(q, j):
            pltpu.make_async_copy(src_hbm.at[cid, e0 + j], srcs[q],
                                  isem[q]).wait()
            pltpu.make_async_copy(dst_hbm.at[e0 + j], dsts[q],
                                  isem[q]).wait()

        for q in range(NIDX):
            pltpu.async_copy(src_hbm.at[cid, e0 + q], srcs[q], isem[q])
            pltpu.async_copy(dst_hbm.at[e0 + q], dsts[q], isem[q])
        for j in range(GA):
            idx_wait(j, j)
            pltpu.async_copy(h_hbm.at[srcs[j]], rows[j], gsem[j])

        UNROLL = NBUF * NIDX // _gcd(NBUF, NIDX)
        assert CPB % UNROLL == 0

        def outer(jo, carry):
            for u in range(UNROLL):
                j = jo * UNROLL + u
                b = u % NBUF
                q = u % NIDX
                pltpu.make_async_copy(h_hbm.at[srcs[q]], rows[b],
                                      gsem[b]).wait()
                pltpu.async_copy(rows[b], msum_sh.at[dsts[q]], ssem[b],
                                 add=True)
                if with_cnt:
                    @pl.when(cid == 0)
                    def _():
                        pltpu.sync_copy(ones_v, cnt_sh.at[dsts[q]], add=True)

                b2 = (u + NBUF - 2) % NBUF
                q2 = (u + NIDX - 2) % NIDX

                @pl.when(j >= 2)
                def _():
                    pltpu.make_async_copy(rows[b2],
                                          msum_sh.at[dsts[q2]],
                                          ssem[b2]).wait()

                @pl.when((j >= 2) & (j + NIDX - 2 < CPB))
                def _():
                    pltpu.async_copy(src_hbm.at[cid, e0 + j + NIDX - 2],
                                     srcs[q2], isem[q2])
                    pltpu.async_copy(dst_hbm.at[e0 + j + NIDX - 2],
                                     dsts[q2], isem[q2])

                bq = (u + GA) % NBUF
                qq = (u + GA) % NIDX

                @pl.when(j + GA < CPB)
                def _():
                    idx_wait(qq, j + GA)
                    pltpu.async_copy(h_hbm.at[srcs[qq]], rows[bq], gsem[bq])
            return carry

        lax.fori_loop(0, CPB // UNROLL, outer, 0)
        pltpu.make_async_copy(rows[(CPB - 2) % NBUF],
                              msum_sh.at[dsts[(CPB - 2) % NIDX]],
                              ssem[(CPB - 2) % NBUF]).wait()
        pltpu.make_async_copy(rows[(CPB - 1) % NBUF],
                              msum_sh.at[dsts[(CPB - 1) % NIDX]],
                              ssem[(CPB - 1) % NBUF]).wait()
        plsc.subcore_barrier()

        pltpu.sync_copy(msum_sh.at[pl.ds(r0, RPT)],
                        msum_out.at[cid, pl.ds(r0, RPT)])
        if with_cnt:
            @pl.when(cid == 0)
            def _():
                pltpu.sync_copy(cnt_sh.at[pl.ds(r0, RPT)],
                                cnt_out.at[pl.ds(r0, RPT)])

    if with_cnt:
        out_type = (jax.ShapeDtypeStruct((NC, NROW, 128), jnp.float32),
                    jax.ShapeDtypeStruct((NROW,), jnp.float32))
    else:
        out_type = jax.ShapeDtypeStruct((NC, NROW, 128), jnp.float32)

    fn = pl.kernel(
        body,
        out_type=out_type,
        mesh=mesh,
        scratch_types=(
            [
                pltpu.VMEM((CK,), jnp.float32),
                pltpu.VMEM_SHARED((NROW, 128), jnp.float32),
                pltpu.VMEM_SHARED((NROW,), jnp.float32),
            ]
            + [pltpu.VMEM((CK,), jnp.int32) for _ in range(NIDX)]
            + [pltpu.VMEM((CK,), jnp.int32) for _ in range(NIDX)]
            + [pltpu.VMEM((CK, 128), jnp.float32) for _ in range(NBUF)]
            + [pltpu.SemaphoreType.DMA for _ in range(NBUF)]
            + [pltpu.SemaphoreType.DMA for _ in range(NBUF)]
            + [pltpu.SemaphoreType.DMA for _ in range(NIDX)]
        ),
    )
    return fn(h2n, srcadj, dstm2, z2, z1)


def _gate_call(h_split, msum, cnt2, last_raw, tmW, tmb, g, be,
               W_out=None, b_out=None):
    final = W_out is not None
    has_last = last_raw is not None

    def body(*refs):
        refs = list(refs)
        h_ref = refs.pop(0)
        ms_ref = refs.pop(0)
        cnt_ref = refs.pop(0)
        last_ref = refs.pop(0) if has_last else None
        w_ref = refs.pop(0)
        b_ref = refs.pop(0)
        g_ref = refs.pop(0)
        be_ref = refs.pop(0)
        if final:
            wo_ref = refs.pop(0)
            bo_ref = refs.pop(0)
            o_ref = refs.pop(0)
        else:
            o_ref = refs.pop(0)
            raw_ref = refs.pop(0)

        h0 = h_ref[0]
        h1 = h_ref[1]
        rinv = 1.0 / jnp.maximum(cnt_ref[...], 1.0)
        m0 = ms_ref[0] * rinv
        m1 = ms_ref[1] * rinv

        w = w_ref[...]
        z = (jnp.dot(h0, w[0:128], preferred_element_type=jnp.float32)
             + jnp.dot(h1, w[128:256], preferred_element_type=jnp.float32)
             + jnp.dot(m0, w[256:384], preferred_element_type=jnp.float32)
             + jnp.dot(m1, w[384:512], preferred_element_type=jnp.float32)
             + b_ref[...])
        z = z - jnp.max(z, axis=-1, keepdims=True)
        ez = jnp.exp(z)
        sm = ez / jnp.sum(ez, axis=-1, keepdims=True)

        r_i = lax.broadcasted_iota(jnp.int32, (GC, GC), 0)
        c_i = lax.broadcasted_iota(jnp.int32, (GC, GC), 1)
        tri = (r_i <= c_i).astype(jnp.float32)
        craw = jnp.dot(sm, tri, preferred_element_type=jnp.float32)
        if has_last:
            last = last_ref[...]
            craw = last + (1.0 - last) * craw

        jr = lax.broadcasted_iota(jnp.int32, (GC, 128), 0)
        cc = lax.broadcasted_iota(jnp.int32, (GC, 128), 1)
        rep0 = (jr == cc // REP).astype(jnp.float32)
        rep1 = (jr == GC // 2 + cc // REP).astype(jnp.float32)
        tm0 = jnp.dot(craw, rep0, preferred_element_type=jnp.float32)
        tm1 = jnp.dot(craw, rep1, preferred_element_type=jnp.float32)

        n0 = h0 * tm0 + m0 * (1.0 - tm0)
        n1 = h1 * tm1 + m1 * (1.0 - tm1)

        s = jnp.sum(n0, axis=-1, keepdims=True) + jnp.sum(n1, axis=-1, keepdims=True)
        mu = s / H
        d0 = n0 - mu
        d1 = n1 - mu
        var = (jnp.sum(d0 * d0, axis=-1, keepdims=True)
               + jnp.sum(d1 * d1, axis=-1, keepdims=True)) / H
        rstd = lax.rsqrt(var + 1e-5)
        gv = g_ref[...]
        bev = be_ref[...]
        f0 = d0 * rstd * gv[:, 0:128] + bev[:, 0:128]
        f1 = d1 * rstd * gv[:, 128:256] + bev[:, 128:256]

        if final:
            wo = wo_ref[...]
            o_ref[...] = (jnp.dot(f0, wo[0:128], preferred_element_type=jnp.float32)
                          + jnp.dot(f1, wo[128:256], preferred_element_type=jnp.float32)
                          + bo_ref[...])
        else:
            o_ref[0] = f0
            o_ref[1] = f1
            raw_ref[...] = craw

    in_specs = [
        pl.BlockSpec((2, BN, 128), lambda i: (0, i, 0)),
        pl.BlockSpec((2, BN, 128), lambda i: (0, i, 0)),
        pl.BlockSpec((BN, 1), lambda i: (i, 0)),
    ]
    args = [h_split, msum, cnt2]
    if has_last:
        in_specs.append(pl.BlockSpec((BN, GC), lambda i: (i, 0)))
        args.append(last_raw)
    in_specs += [
        pl.BlockSpec((2 * H, GC), lambda i: (0, 0)),
        pl.BlockSpec((1, GC), lambda i: (0, 0)),
        pl.BlockSpec((1, H), lambda i: (0, 0)),
        pl.BlockSpec((1, H), lambda i: (0, 0)),
    ]
    args += [tmW, tmb.reshape(1, GC), g.reshape(1, H), be.reshape(1, H)]
    if final:
        in_specs += [
            pl.BlockSpec((H, DOUT), lambda i: (0, 0)),
            pl.BlockSpec((1, DOUT), lambda i: (0, 0)),
        ]
        args += [W_out, b_out.reshape(1, DOUT)]
        out_specs = pl.BlockSpec((BN, DOUT), lambda i: (i, 0))
        out_shape = jax.ShapeDtypeStruct((N, DOUT), jnp.float32)
    else:
        out_specs = (
            pl.BlockSpec((2, BN, 128), lambda i: (0, i, 0)),
            pl.BlockSpec((BN, GC), lambda i: (i, 0)),
        )
        out_shape = (
            jax.ShapeDtypeStruct((2, N, 128), jnp.float32),
            jax.ShapeDtypeStruct((N, GC), jnp.float32),
        )

    return pl.pallas_call(
        body,
        grid=(N // BN,),
        in_specs=in_specs,
        out_specs=out_specs,
        out_shape=out_shape,
    )(*args)


@jax.jit
def kernel(x, edge_index, W_in, b_in, g_in, beta_in,
           tmW0, tmb0, g0, beta0, tmW1, tmb1, g1, beta1, W_out, b_out):
    src = edge_index[0]
    dst = edge_index[1]
    pad = jnp.zeros((EPAD - E,), jnp.int32)
    srcp = jnp.concatenate([src, pad])
    dstp = jnp.concatenate([dst, pad])

    dstm2, srcadj = _edge_mask_call(srcp.reshape(EB, 128),
                                    dstp.reshape(EB, 128))

    h_split = _input_call(x, W_in, b_in, g_in, beta_in)

    z2 = jnp.zeros((RPT, 128), jnp.float32)
    z1 = jnp.zeros((RPT,), jnp.float32)

    srcc = srcadj.reshape(2, ECH, CK)
    dstc = dstm2.reshape(ECH, CK)

    msum1, cnt = _sc_segsum(h_split.reshape(2 * N, 128), srcc, dstc,
                            z2, z1, with_cnt=True)
    cnt2 = cnt.reshape(NROW, 1)

    h_split2, raw1 = _gate_call(h_split, msum1, cnt2, None,
                                tmW0, tmb0, g0, beta0)

    msum2 = _sc_segsum(h_split2.reshape(2 * N, 128), srcc, dstc,
                       z2, z1, with_cnt=False)

    return _gate_call(h_split2, msum2, cnt2, raw1,
                      tmW1, tmb1, g1, beta1, W_out=W_out, b_out=b_out)


# speedup vs baseline: 4.9357x; 1.1598x over previous
import functools
from math import gcd as _gcd

import jax
import jax.numpy as jnp
from jax import lax
from jax.experimental import pallas as pl
from jax.experimental.pallas import tpu as pltpu
from jax.experimental.pallas import tpu_sc as plsc

N = 10000
E = 320000
DIN = 128
H = 256
DOUT = 128
GC = 64
REP = 4

NC = 2
NS = 16
CK = 64
GB = 4
GF = 2
GA = 2
NIDX = 8
CPB = 320
EPAD = NS * CPB * CK
EB = EPAD // 128
ECH = EPAD // CK
TRASH = N
NROW = 10240
RPT = NROW // NS

BN = 1000


def _edge_mask_call(src2, dst2):
    def body(s_ref, d_ref, o_ref, sa_ref):
        s = s_ref[...]
        d = d_ref[...]
        o_ref[...] = jnp.where(s != d, d, TRASH)
        sa_ref[0] = s
        sa_ref[1] = s + N

    return pl.pallas_call(
        body,
        out_shape=(jax.ShapeDtypeStruct((EB, 128), jnp.int32),
                   jax.ShapeDtypeStruct((2, EB, 128), jnp.int32)),
    )(src2, dst2)


def _perm_mat():
    ri = lax.broadcasted_iota(jnp.int32, (128, 128), 0)
    ci = lax.broadcasted_iota(jnp.int32, (128, 128), 1)
    rmap = 32 * (ci // 32) + 16 * (ci % 2) + (ci % 32) // 2
    return (ri == rmap).astype(jnp.float32)


def _input_call(x, W_in, b_in, g_in, beta_in):
    def body(x_ref, w_ref, b_ref, g_ref, be_ref, o_ref, ob_ref):
        t = jnp.dot(x_ref[...], w_ref[...], preferred_element_type=jnp.float32)
        t = jnp.maximum(t + b_ref[...], 0.0)
        mu = jnp.mean(t, axis=-1, keepdims=True)
        d = t - mu
        var = jnp.mean(d * d, axis=-1, keepdims=True)
        hn = d * lax.rsqrt(var + 1e-5) * g_ref[...] + be_ref[...]
        h0 = hn[:, :128]
        h1 = hn[:, 128:]
        o_ref[0] = h0
        o_ref[1] = h1
        P = _perm_mat()
        ob_ref[0] = jnp.dot(h0, P, preferred_element_type=jnp.float32
                            ).astype(jnp.bfloat16)
        ob_ref[1] = jnp.dot(h1, P, preferred_element_type=jnp.float32
                            ).astype(jnp.bfloat16)

    return pl.pallas_call(
        body,
        grid=(N // BN,),
        in_specs=[
            pl.BlockSpec((BN, DIN), lambda i: (i, 0)),
            pl.BlockSpec((DIN, H), lambda i: (0, 0)),
            pl.BlockSpec((1, H), lambda i: (0, 0)),
            pl.BlockSpec((1, H), lambda i: (0, 0)),
            pl.BlockSpec((1, H), lambda i: (0, 0)),
        ],
        out_specs=(pl.BlockSpec((2, BN, 128), lambda i: (0, i, 0)),
                   pl.BlockSpec((2, BN, 128), lambda i: (0, i, 0))),
        out_shape=(jax.ShapeDtypeStruct((2, N, 128), jnp.float32),
                   jax.ShapeDtypeStruct((2, N, 128), jnp.bfloat16)),
    )(x, W_in, b_in.reshape(1, H), g_in.reshape(1, H), beta_in.reshape(1, H))


def _sc_segsum(h2n, srcadj, dstm2, z2, z1, with_cnt):
    mesh = plsc.VectorSubcoreMesh(core_axis_name="c", subcore_axis_name="s")

    def body(h_hbm, src_hbm, dst_hbm, z2_hbm, z1_hbm, *rest):
        if with_cnt:
            msum_out, cnt_out = rest[0], rest[1]
            rest = rest[2:]
        else:
            msum_out = rest[0]
            cnt_out = None
            rest = rest[1:]
        ones_v, msum_sh, cnt_sh = rest[:3]
        rest = rest[3:]
        srcs = rest[0:NIDX]
        dsts = rest[NIDX:2 * NIDX]
        rows_bf = rest[2 * NIDX:2 * NIDX + GB]
        rest = rest[2 * NIDX + GB:]
        rows_f = rest[0:GF]
        rest = rest[GF:]
        gsem = rest[0:GB]
        ssem = rest[GB:GB + GF]
        isem = rest[GB + GF:GB + GF + NIDX]

        cid = lax.axis_index("c")
        sid = lax.axis_index("s")
        r0 = sid * RPT
        e0 = sid * CPB

        pltpu.sync_copy(z2_hbm, msum_sh.at[pl.ds(r0, RPT)])
        if with_cnt:
            pltpu.sync_copy(z1_hbm, cnt_sh.at[pl.ds(r0, RPT)])
            for i in range(CK // 16):
                ones_v[pl.ds(i * 16, 16)] = jnp.ones((16,), jnp.float32)
        plsc.subcore_barrier()

        def idx_wait(q, j):
            pltpu.make_async_copy(src_hbm.at[cid, e0 + j], srcs[q],
                                  isem[q]).wait()
            pltpu.make_async_copy(dst_hbm.at[e0 + j], dsts[q],
                                  isem[q]).wait()

        for q in range(NIDX):
            pltpu.async_copy(src_hbm.at[cid, e0 + q], srcs[q], isem[q])
            pltpu.async_copy(dst_hbm.at[e0 + q], dsts[q], isem[q])
        for j in range(GA):
            idx_wait(j, j)
            pltpu.async_copy(h_hbm.at[srcs[j]], rows_bf[j], gsem[j])

        UNROLL = NIDX
        assert CPB % UNROLL == 0 and UNROLL % GB == 0 and UNROLL % GF == 0

        def outer(jo, carry):
            for u in range(UNROLL):
                j = jo * UNROLL + u
                bb = u % GB
                bf = u % GF
                q = u % NIDX
                q2 = (u + NIDX - 2) % NIDX
                pltpu.make_async_copy(h_hbm.at[srcs[q]], rows_bf[bb],
                                      gsem[bb]).wait()

                @pl.when(j >= 2)
                def _():
                    pltpu.make_async_copy(rows_f[bf],
                                          msum_sh.at[dsts[q2]],
                                          ssem[bf]).wait()

                @pl.when((j >= 2) & (j + NIDX - 2 < CPB))
                def _():
                    pltpu.async_copy(src_hbm.at[cid, e0 + j + NIDX - 2],
                                     srcs[q2], isem[q2])
                    pltpu.async_copy(dst_hbm.at[e0 + j + NIDX - 2],
                                     dsts[q2], isem[q2])

                bq = (u + GA) % GB
                qq = (u + GA) % NIDX

                @pl.when(j + GA < CPB)
                def _():
                    idx_wait(qq, j + GA)
                    pltpu.async_copy(h_hbm.at[srcs[qq]], rows_bf[bq],
                                     gsem[bq])

                for r in range(CK):
                    for k in range(4):
                        w = rows_bf[bb][r, pl.ds(k * 16, 16)]
                        lo = lax.bitcast_convert_type(w << 16, jnp.float32)
                        hi = lax.bitcast_convert_type(
                            w & jnp.int32(-65536), jnp.float32)
                        rows_f[bf][r, pl.ds(k * 32, 16)] = lo
                        rows_f[bf][r, pl.ds(k * 32 + 16, 16)] = hi

                pltpu.async_copy(rows_f[bf], msum_sh.at[dsts[q]], ssem[bf],
                                 add=True)
                if with_cnt:
                    @pl.when(cid == 0)
                    def _():
                        pltpu.sync_copy(ones_v, cnt_sh.at[dsts[q]], add=True)
            return carry

        lax.fori_loop(0, CPB // UNROLL, outer, 0)
        pltpu.make_async_copy(rows_f[(CPB - 2) % GF],
                              msum_sh.at[dsts[(CPB - 2) % NIDX]],
                              ssem[(CPB - 2) % GF]).wait()
        pltpu.make_async_copy(rows_f[(CPB - 1) % GF],
                              msum_sh.at[dsts[(CPB - 1) % NIDX]],
                              ssem[(CPB - 1) % GF]).wait()
        plsc.subcore_barrier()

        pltpu.sync_copy(msum_sh.at[pl.ds(r0, RPT)],
                        msum_out.at[cid, pl.ds(r0, RPT)])
        if with_cnt:
            @pl.when(cid == 0)
            def _():
                pltpu.sync_copy(cnt_sh.at[pl.ds(r0, RPT)],
                                cnt_out.at[pl.ds(r0, RPT)])

    if with_cnt:
        out_type = (jax.ShapeDtypeStruct((NC, NROW, 128), jnp.float32),
                    jax.ShapeDtypeStruct((NROW,), jnp.float32))
    else:
        out_type = jax.ShapeDtypeStruct((NC, NROW, 128), jnp.float32)

    fn = pl.kernel(
        body,
        out_type=out_type,
        mesh=mesh,
        compiler_params=pltpu.CompilerParams(use_tc_tiling_on_sc=False),
        scratch_types=(
            [
                pltpu.VMEM((CK,), jnp.float32),
                pltpu.VMEM_SHARED((NROW, 128), jnp.float32),
                pltpu.VMEM_SHARED((NROW,), jnp.float32),
            ]
            + [pltpu.VMEM((CK,), jnp.int32) for _ in range(NIDX)]
            + [pltpu.VMEM((CK,), jnp.int32) for _ in range(NIDX)]
            + [pltpu.VMEM((CK, 64), jnp.int32) for _ in range(GB)]
            + [pltpu.VMEM((CK, 128), jnp.float32) for _ in range(GF)]
            + [pltpu.SemaphoreType.DMA for _ in range(GB)]
            + [pltpu.SemaphoreType.DMA for _ in range(GF)]
            + [pltpu.SemaphoreType.DMA for _ in range(NIDX)]
        ),
    )
    return fn(h2n, srcadj, dstm2, z2, z1)


def _gate_call(h_split, msum, cnt2, last_raw, tmW, tmb, g, be,
               W_out=None, b_out=None):
    final = W_out is not None
    has_last = last_raw is not None

    def body(*refs):
        refs = list(refs)
        h_ref = refs.pop(0)
        ms_ref = refs.pop(0)
        cnt_ref = refs.pop(0)
        last_ref = refs.pop(0) if has_last else None
        w_ref = refs.pop(0)
        b_ref = refs.pop(0)
        g_ref = refs.pop(0)
        be_ref = refs.pop(0)
        if final:
            wo_ref = refs.pop(0)
            bo_ref = refs.pop(0)
            o_ref = refs.pop(0)
        else:
            o_ref = refs.pop(0)
            raw_ref = refs.pop(0)
            ob_ref = refs.pop(0)

        h0 = h_ref[0]
        h1 = h_ref[1]
        rinv = 1.0 / jnp.maximum(cnt_ref[...], 1.0)
        m0 = ms_ref[0] * rinv
        m1 = ms_ref[1] * rinv

        w = w_ref[...]
        z = (jnp.dot(h0, w[0:128], preferred_element_type=jnp.float32)
             + jnp.dot(h1, w[128:256], preferred_element_type=jnp.float32)
             + jnp.dot(m0, w[256:384], preferred_element_type=jnp.float32)
             + jnp.dot(m1, w[384:512], preferred_element_type=jnp.float32)
             + b_ref[...])
        z = z - jnp.max(z, axis=-1, keepdims=True)
        ez = jnp.exp(z)
        sm = ez / jnp.sum(ez, axis=-1, keepdims=True)

        r_i = lax.broadcasted_iota(jnp.int32, (GC, GC), 0)
        c_i = lax.broadcasted_iota(jnp.int32, (GC, GC), 1)
        tri = (r_i <= c_i).astype(jnp.float32)
        craw = jnp.dot(sm, tri, preferred_element_type=jnp.float32)
        if has_last:
            last = last_ref[...]
            craw = last + (1.0 - last) * craw

        jr = lax.broadcasted_iota(jnp.int32, (GC, 128), 0)
        cc = lax.broadcasted_iota(jnp.int32, (GC, 128), 1)
        rep0 = (jr == cc // REP).astype(jnp.float32)
        rep1 = (jr == GC // 2 + cc // REP).astype(jnp.float32)
        tm0 = jnp.dot(craw, rep0, preferred_element_type=jnp.float32)
        tm1 = jnp.dot(craw, rep1, preferred_element_type=jnp.float32)

        n0 = h0 * tm0 + m0 * (1.0 - tm0)
        n1 = h1 * tm1 + m1 * (1.0 - tm1)

        s = jnp.sum(n0, axis=-1, keepdims=True) + jnp.sum(n1, axis=-1, keepdims=True)
        mu = s / H
        d0 = n0 - mu
        d1 = n1 - mu
        var = (jnp.sum(d0 * d0, axis=-1, keepdims=True)
               + jnp.sum(d1 * d1, axis=-1, keepdims=True)) / H
        rstd = lax.rsqrt(var + 1e-5)
        gv = g_ref[...]
        bev = be_ref[...]
        f0 = d0 * rstd * gv[:, 0:128] + bev[:, 0:128]
        f1 = d1 * rstd * gv[:, 128:256] + bev[:, 128:256]

        if final:
            wo = wo_ref[...]
            o_ref[...] = (jnp.dot(f0, wo[0:128], preferred_element_type=jnp.float32)
                          + jnp.dot(f1, wo[128:256], preferred_element_type=jnp.float32)
                          + bo_ref[...])
        else:
            o_ref[0] = f0
            o_ref[1] = f1
            raw_ref[...] = craw
            P = _perm_mat()
            ob_ref[0] = jnp.dot(f0, P, preferred_element_type=jnp.float32
                                ).astype(jnp.bfloat16)
            ob_ref[1] = jnp.dot(f1, P, preferred_element_type=jnp.float32
                                ).astype(jnp.bfloat16)

    in_specs = [
        pl.BlockSpec((2, BN, 128), lambda i: (0, i, 0)),
        pl.BlockSpec((2, BN, 128), lambda i: (0, i, 0)),
        pl.BlockSpec((BN, 1), lambda i: (i, 0)),
    ]
    args = [h_split, msum, cnt2]
    if has_last:
        in_specs.append(pl.BlockSpec((BN, GC), lambda i: (i, 0)))
        args.append(last_raw)
    in_specs += [
        pl.BlockSpec((2 * H, GC), lambda i: (0, 0)),
        pl.BlockSpec((1, GC), lambda i: (0, 0)),
        pl.BlockSpec((1, H), lambda i: (0, 0)),
        pl.BlockSpec((1, H), lambda i: (0, 0)),
    ]
    args += [tmW, tmb.reshape(1, GC), g.reshape(1, H), be.reshape(1, H)]
    if final:
        in_specs += [
            pl.BlockSpec((H, DOUT), lambda i: (0, 0)),
            pl.BlockSpec((1, DOUT), lambda i: (0, 0)),
        ]
        args += [W_out, b_out.reshape(1, DOUT)]
        out_specs = pl.BlockSpec((BN, DOUT), lambda i: (i, 0))
        out_shape = jax.ShapeDtypeStruct((N, DOUT), jnp.float32)
    else:
        out_specs = (
            pl.BlockSpec((2, BN, 128), lambda i: (0, i, 0)),
            pl.BlockSpec((BN, GC), lambda i: (i, 0)),
            pl.BlockSpec((2, BN, 128), lambda i: (0, i, 0)),
        )
        out_shape = (
            jax.ShapeDtypeStruct((2, N, 128), jnp.float32),
            jax.ShapeDtypeStruct((N, GC), jnp.float32),
            jax.ShapeDtypeStruct((2, N, 128), jnp.bfloat16),
        )

    return pl.pallas_call(
        body,
        grid=(N // BN,),
        in_specs=in_specs,
        out_specs=out_specs,
        out_shape=out_shape,
    )(*args)


@jax.jit
def kernel(x, edge_index, W_in, b_in, g_in, beta_in,
           tmW0, tmb0, g0, beta0, tmW1, tmb1, g1, beta1, W_out, b_out):
    src = edge_index[0]
    dst = edge_index[1]
    pad = jnp.zeros((EPAD - E,), jnp.int32)
    srcp = jnp.concatenate([src, pad])
    dstp = jnp.concatenate([dst, pad])

    dstm2, srcadj = _edge_mask_call(srcp.reshape(EB, 128),
                                    dstp.reshape(EB, 128))

    h_split, hb = _input_call(x, W_in, b_in, g_in, beta_in)

    z2 = jnp.zeros((RPT, 128), jnp.float32)
    z1 = jnp.zeros((RPT,), jnp.float32)

    srcc = srcadj.reshape(2, ECH, CK)
    dstc = dstm2.reshape(ECH, CK)

    hbw = lax.bitcast_convert_type(hb.reshape(2 * N, 64, 2), jnp.int32)
    msum1, cnt = _sc_segsum(hbw, srcc, dstc,
                            z2, z1, with_cnt=True)
    cnt2 = cnt.reshape(NROW, 1)

    h_split2, raw1, hb2 = _gate_call(h_split, msum1, cnt2, None,
                                     tmW0, tmb0, g0, beta0)

    hbw2 = lax.bitcast_convert_type(hb2.reshape(2 * N, 64, 2), jnp.int32)
    msum2 = _sc_segsum(hbw2, srcc, dstc,
                       z2, z1, with_cnt=False)

    return _gate_call(h_split2, msum2, cnt2, raw1,
                      tmW1, tmb1, g1, beta1, W_out=W_out, b_out=b_out)


# speedup vs baseline: 5.0475x; 1.0227x over previous
import functools
from math import gcd as _gcd

import jax
import jax.numpy as jnp
from jax import lax
from jax.experimental import pallas as pl
from jax.experimental.pallas import tpu as pltpu
from jax.experimental.pallas import tpu_sc as plsc

N = 10000
E = 320000
DIN = 128
H = 256
DOUT = 128
GC = 64
REP = 4

NC = 2
NS = 16
CK = 64
GB = 5
GF = 2
GA = 3
NIDX = 10
CPB = 320
EPAD = NS * CPB * CK
EB = EPAD // 128
ECH = EPAD // CK
TRASH = N
NROW = 10240
RPT = NROW // NS

BN = 1000


def _edge_mask_call(src2, dst2):
    def body(s_ref, d_ref, o_ref, sa_ref):
        s = s_ref[...]
        d = d_ref[...]
        o_ref[...] = jnp.where(s != d, d, TRASH)
        sa_ref[0] = s
        sa_ref[1] = s + N

    return pl.pallas_call(
        body,
        out_shape=(jax.ShapeDtypeStruct((EB, 128), jnp.int32),
                   jax.ShapeDtypeStruct((2, EB, 128), jnp.int32)),
    )(src2, dst2)


def _perm_mat():
    ri = lax.broadcasted_iota(jnp.int32, (128, 128), 0)
    ci = lax.broadcasted_iota(jnp.int32, (128, 128), 1)
    rmap = 32 * (ci // 32) + 16 * (ci % 2) + (ci % 32) // 2
    return (ri == rmap).astype(jnp.float32)


def _input_call(x, W_in, b_in, g_in, beta_in):
    def body(x_ref, w_ref, b_ref, g_ref, be_ref, o_ref, ob_ref):
        t = jnp.dot(x_ref[...], w_ref[...], preferred_element_type=jnp.float32)
        t = jnp.maximum(t + b_ref[...], 0.0)
        mu = jnp.mean(t, axis=-1, keepdims=True)
        d = t - mu
        var = jnp.mean(d * d, axis=-1, keepdims=True)
        hn = d * lax.rsqrt(var + 1e-5) * g_ref[...] + be_ref[...]
        h0 = hn[:, :128]
        h1 = hn[:, 128:]
        o_ref[0] = h0
        o_ref[1] = h1
        P = _perm_mat()
        ob_ref[0] = jnp.dot(h0, P, preferred_element_type=jnp.float32
                            ).astype(jnp.bfloat16)
        ob_ref[1] = jnp.dot(h1, P, preferred_element_type=jnp.float32
                            ).astype(jnp.bfloat16)

    return pl.pallas_call(
        body,
        grid=(N // BN,),
        in_specs=[
            pl.BlockSpec((BN, DIN), lambda i: (i, 0)),
            pl.BlockSpec((DIN, H), lambda i: (0, 0)),
            pl.BlockSpec((1, H), lambda i: (0, 0)),
            pl.BlockSpec((1, H), lambda i: (0, 0)),
            pl.BlockSpec((1, H), lambda i: (0, 0)),
        ],
        out_specs=(pl.BlockSpec((2, BN, 128), lambda i: (0, i, 0)),
                   pl.BlockSpec((2, BN, 128), lambda i: (0, i, 0))),
        out_shape=(jax.ShapeDtypeStruct((2, N, 128), jnp.float32),
                   jax.ShapeDtypeStruct((2, N, 128), jnp.bfloat16)),
    )(x, W_in, b_in.reshape(1, H), g_in.reshape(1, H), beta_in.reshape(1, H))


def _sc_segsum(h2n, srcadj, dstm2, z2, z1, with_cnt):
    mesh = plsc.VectorSubcoreMesh(core_axis_name="c", subcore_axis_name="s")

    def body(h_hbm, src_hbm, dst_hbm, z2_hbm, z1_hbm, *rest):
        if with_cnt:
            msum_out, cnt_out = rest[0], rest[1]
            rest = rest[2:]
        else:
            msum_out = rest[0]
            cnt_out = None
            rest = rest[1:]
        ones_v, msum_sh, cnt_sh = rest[:3]
        rest = rest[3:]
        srcs = rest[0:NIDX]
        dsts = rest[NIDX:2 * NIDX]
        rows_bf = rest[2 * NIDX:2 * NIDX + GB]
        rest = rest[2 * NIDX + GB:]
        rows_f = rest[0:GF]
        rest = rest[GF:]
        gsem = rest[0:GB]
        ssem = rest[GB:GB + GF]
        csem = rest[GB + GF:GB + GF + 2]
        isem = rest[GB + GF + 2:GB + GF + 2 + NIDX]

        cid = lax.axis_index("c")
        sid = lax.axis_index("s")
        r0 = sid * RPT
        e0 = sid * CPB

        pltpu.sync_copy(z2_hbm, msum_sh.at[pl.ds(r0, RPT)])
        if with_cnt:
            pltpu.sync_copy(z1_hbm, cnt_sh.at[pl.ds(r0, RPT)])
            for i in range(CK // 16):
                ones_v[pl.ds(i * 16, 16)] = jnp.ones((16,), jnp.float32)
        plsc.subcore_barrier()

        def idx_wait(q, j):
            pltpu.make_async_copy(src_hbm.at[cid, e0 + j], srcs[q],
                                  isem[q]).wait()
            pltpu.make_async_copy(dst_hbm.at[e0 + j], dsts[q],
                                  isem[q]).wait()

        for q in range(NIDX):
            pltpu.async_copy(src_hbm.at[cid, e0 + q], srcs[q], isem[q])
            pltpu.async_copy(dst_hbm.at[e0 + q], dsts[q], isem[q])
        for j in range(GA):
            idx_wait(j, j)
            pltpu.async_copy(h_hbm.at[srcs[j]], rows_bf[j], gsem[j])

        UNROLL = NIDX
        assert CPB % UNROLL == 0 and UNROLL % GB == 0 and UNROLL % GF == 0

        def outer(jo, carry):
            for u in range(UNROLL):
                j = jo * UNROLL + u
                bb = u % GB
                bf = u % GF
                q = u % NIDX
                q2 = (u + NIDX - 2) % NIDX
                pltpu.make_async_copy(h_hbm.at[srcs[q]], rows_bf[bb],
                                      gsem[bb]).wait()

                @pl.when(j >= 2)
                def _():
                    pltpu.make_async_copy(rows_f[bf],
                                          msum_sh.at[dsts[q2]],
                                          ssem[bf]).wait()
                    if with_cnt:
                        @pl.when(cid == 0)
                        def _():
                            pltpu.make_async_copy(ones_v,
                                                  cnt_sh.at[dsts[q2]],
                                                  csem[bf]).wait()

                @pl.when((j >= 2) & (j + NIDX - 2 < CPB))
                def _():
                    pltpu.async_copy(src_hbm.at[cid, e0 + j + NIDX - 2],
                                     srcs[q2], isem[q2])
                    pltpu.async_copy(dst_hbm.at[e0 + j + NIDX - 2],
                                     dsts[q2], isem[q2])

                bq = (u + GA) % GB
                qq = (u + GA) % NIDX

                @pl.when(j + GA < CPB)
                def _():
                    idx_wait(qq, j + GA)
                    pltpu.async_copy(h_hbm.at[srcs[qq]], rows_bf[bq],
                                     gsem[bq])

                for r in range(CK):
                    for k in range(4):
                        w = rows_bf[bb][r, pl.ds(k * 16, 16)]
                        lo = lax.bitcast_convert_type(w << 16, jnp.float32)
                        hi = lax.bitcast_convert_type(
                            w & jnp.int32(-65536), jnp.float32)
                        rows_f[bf][r, pl.ds(k * 32, 16)] = lo
                        rows_f[bf][r, pl.ds(k * 32 + 16, 16)] = hi

                pltpu.async_copy(rows_f[bf], msum_sh.at[dsts[q]], ssem[bf],
                                 add=True)
                if with_cnt:
                    @pl.when(cid == 0)
                    def _():
                        pltpu.async_copy(ones_v, cnt_sh.at[dsts[q]],
                                         csem[bf], add=True)
            return carry

        lax.fori_loop(0, CPB // UNROLL, outer, 0)
        pltpu.make_async_copy(rows_f[(CPB - 2) % GF],
                              msum_sh.at[dsts[(CPB - 2) % NIDX]],
                              ssem[(CPB - 2) % GF]).wait()
        pltpu.make_async_copy(rows_f[(CPB - 1) % GF],
                              msum_sh.at[dsts[(CPB - 1) % NIDX]],
                              ssem[(CPB - 1) % GF]).wait()
        if with_cnt:
            @pl.when(cid == 0)
            def _():
                pltpu.make_async_copy(ones_v,
                                      cnt_sh.at[dsts[(CPB - 2) % NIDX]],
                                      csem[(CPB - 2) % GF]).wait()
                pltpu.make_async_copy(ones_v,
                                      cnt_sh.at[dsts[(CPB - 1) % NIDX]],
                                      csem[(CPB - 1) % GF]).wait()
        plsc.subcore_barrier()

        pltpu.sync_copy(msum_sh.at[pl.ds(r0, RPT)],
                        msum_out.at[cid, pl.ds(r0, RPT)])
        if with_cnt:
            @pl.when(cid == 0)
            def _():
                pltpu.sync_copy(cnt_sh.at[pl.ds(r0, RPT)],
                                cnt_out.at[pl.ds(r0, RPT)])

    if with_cnt:
        out_type = (jax.ShapeDtypeStruct((NC, NROW, 128), jnp.float32),
                    jax.ShapeDtypeStruct((NROW,), jnp.float32))
    else:
        out_type = jax.ShapeDtypeStruct((NC, NROW, 128), jnp.float32)

    fn = pl.kernel(
        body,
        out_type=out_type,
        mesh=mesh,
        compiler_params=pltpu.CompilerParams(use_tc_tiling_on_sc=False),
        scratch_types=(
            [
                pltpu.VMEM((CK,), jnp.float32),
                pltpu.VMEM_SHARED((NROW, 128), jnp.float32),
                pltpu.VMEM_SHARED((NROW,), jnp.float32),
            ]
            + [pltpu.VMEM((CK,), jnp.int32) for _ in range(NIDX)]
            + [pltpu.VMEM((CK,), jnp.int32) for _ in range(NIDX)]
            + [pltpu.VMEM((CK, 64), jnp.int32) for _ in range(GB)]
            + [pltpu.VMEM((CK, 128), jnp.float32) for _ in range(GF)]
            + [pltpu.SemaphoreType.DMA for _ in range(GB)]
            + [pltpu.SemaphoreType.DMA for _ in range(GF)]
            + [pltpu.SemaphoreType.DMA for _ in range(2)]
            + [pltpu.SemaphoreType.DMA for _ in range(NIDX)]
        ),
    )
    return fn(h2n, srcadj, dstm2, z2, z1)


def _gate_call(h_split, msum, cnt2, last_raw, tmW, tmb, g, be,
               W_out=None, b_out=None):
    final = W_out is not None
    has_last = last_raw is not None

    def body(*refs):
        refs = list(refs)
        h_ref = refs.pop(0)
        ms_ref = refs.pop(0)
        cnt_ref = refs.pop(0)
        last_ref = refs.pop(0) if has_last else None
        w_ref = refs.pop(0)
        b_ref = refs.pop(0)
        g_ref = refs.pop(0)
        be_ref = refs.pop(0)
        if final:
            wo_ref = refs.pop(0)
            bo_ref = refs.pop(0)
            o_ref = refs.pop(0)
        else:
            o_ref = refs.pop(0)
            raw_ref = refs.pop(0)
            ob_ref = refs.pop(0)

        h0 = h_ref[0]
        h1 = h_ref[1]
        rinv = 1.0 / jnp.maximum(cnt_ref[...], 1.0)
        m0 = ms_ref[0] * rinv
        m1 = ms_ref[1] * rinv

        w = w_ref[...]
        z = (jnp.dot(h0, w[0:128], preferred_element_type=jnp.float32)
             + jnp.dot(h1, w[128:256], preferred_element_type=jnp.float32)
             + jnp.dot(m0, w[256:384], preferred_element_type=jnp.float32)
             + jnp.dot(m1, w[384:512], preferred_element_type=jnp.float32)
             + b_ref[...])
        z = z - jnp.max(z, axis=-1, keepdims=True)
        ez = jnp.exp(z)
        sm = ez / jnp.sum(ez, axis=-1, keepdims=True)

        r_i = lax.broadcasted_iota(jnp.int32, (GC, GC), 0)
        c_i = lax.broadcasted_iota(jnp.int32, (GC, GC), 1)
        tri = (r_i <= c_i).astype(jnp.float32)
        craw = jnp.dot(sm, tri, preferred_element_type=jnp.float32)
        if has_last:
            last = last_ref[...]
            craw = last + (1.0 - last) * craw

        jr = lax.broadcasted_iota(jnp.int32, (GC, 128), 0)
        cc = lax.broadcasted_iota(jnp.int32, (GC, 128), 1)
        rep0 = (jr == cc // REP).astype(jnp.float32)
        rep1 = (jr == GC // 2 + cc // REP).astype(jnp.float32)
        tm0 = jnp.dot(craw, rep0, preferred_element_type=jnp.float32)
        tm1 = jnp.dot(craw, rep1, preferred_element_type=jnp.float32)

        n0 = h0 * tm0 + m0 * (1.0 - tm0)
        n1 = h1 * tm1 + m1 * (1.0 - tm1)

        s = jnp.sum(n0, axis=-1, keepdims=True) + jnp.sum(n1, axis=-1, keepdims=True)
        mu = s / H
        d0 = n0 - mu
        d1 = n1 - mu
        var = (jnp.sum(d0 * d0, axis=-1, keepdims=True)
               + jnp.sum(d1 * d1, axis=-1, keepdims=True)) / H
        rstd = lax.rsqrt(var + 1e-5)
        gv = g_ref[...]
        bev = be_ref[...]
        f0 = d0 * rstd * gv[:, 0:128] + bev[:, 0:128]
        f1 = d1 * rstd * gv[:, 128:256] + bev[:, 128:256]

        if final:
            wo = wo_ref[...]
            o_ref[...] = (jnp.dot(f0, wo[0:128], preferred_element_type=jnp.float32)
                          + jnp.dot(f1, wo[128:256], preferred_element_type=jnp.float32)
                          + bo_ref[...])
        else:
            o_ref[0] = f0
            o_ref[1] = f1
            raw_ref[...] = craw
            P = _perm_mat()
            ob_ref[0] = jnp.dot(f0, P, preferred_element_type=jnp.float32
                                ).astype(jnp.bfloat16)
            ob_ref[1] = jnp.dot(f1, P, preferred_element_type=jnp.float32
                                ).astype(jnp.bfloat16)

    in_specs = [
        pl.BlockSpec((2, BN, 128), lambda i: (0, i, 0)),
        pl.BlockSpec((2, BN, 128), lambda i: (0, i, 0)),
        pl.BlockSpec((BN, 1), lambda i: (i, 0)),
    ]
    args = [h_split, msum, cnt2]
    if has_last:
        in_specs.append(pl.BlockSpec((BN, GC), lambda i: (i, 0)))
        args.append(last_raw)
    in_specs += [
        pl.BlockSpec((2 * H, GC), lambda i: (0, 0)),
        pl.BlockSpec((1, GC), lambda i: (0, 0)),
        pl.BlockSpec((1, H), lambda i: (0, 0)),
        pl.BlockSpec((1, H), lambda i: (0, 0)),
    ]
    args += [tmW, tmb.reshape(1, GC), g.reshape(1, H), be.reshape(1, H)]
    if final:
        in_specs += [
            pl.BlockSpec((H, DOUT), lambda i: (0, 0)),
            pl.BlockSpec((1, DOUT), lambda i: (0, 0)),
        ]
        args += [W_out, b_out.reshape(1, DOUT)]
        out_specs = pl.BlockSpec((BN, DOUT), lambda i: (i, 0))
        out_shape = jax.ShapeDtypeStruct((N, DOUT), jnp.float32)
    else:
        out_specs = (
            pl.BlockSpec((2, BN, 128), lambda i: (0, i, 0)),
            pl.BlockSpec((BN, GC), lambda i: (i, 0)),
            pl.BlockSpec((2, BN, 128), lambda i: (0, i, 0)),
        )
        out_shape = (
            jax.ShapeDtypeStruct((2, N, 128), jnp.float32),
            jax.ShapeDtypeStruct((N, GC), jnp.float32),
            jax.ShapeDtypeStruct((2, N, 128), jnp.bfloat16),
        )

    return pl.pallas_call(
        body,
        grid=(N // BN,),
        in_specs=in_specs,
        out_specs=out_specs,
        out_shape=out_shape,
    )(*args)


@jax.jit
def kernel(x, edge_index, W_in, b_in, g_in, beta_in,
           tmW0, tmb0, g0, beta0, tmW1, tmb1, g1, beta1, W_out, b_out):
    src = edge_index[0]
    dst = edge_index[1]
    pad = jnp.zeros((EPAD - E,), jnp.int32)
    srcp = jnp.concatenate([src, pad])
    dstp = jnp.concatenate([dst, pad])

    dstm2, srcadj = _edge_mask_call(srcp.reshape(EB, 128),
                                    dstp.reshape(EB, 128))

    h_split, hb = _input_call(x, W_in, b_in, g_in, beta_in)

    z2 = jnp.zeros((RPT, 128), jnp.float32)
    z1 = jnp.zeros((RPT,), jnp.float32)

    srcc = srcadj.reshape(2, ECH, CK)
    dstc = dstm2.reshape(ECH, CK)

    hbw = lax.bitcast_convert_type(hb.reshape(2 * N, 64, 2), jnp.int32)
    msum1, cnt = _sc_segsum(hbw, srcc, dstc,
                            z2, z1, with_cnt=True)
    cnt2 = cnt.reshape(NROW, 1)

    h_split2, raw1, hb2 = _gate_call(h_split, msum1, cnt2, None,
                                     tmW0, tmb0, g0, beta0)

    hbw2 = lax.bitcast_convert_type(hb2.reshape(2 * N, 64, 2), jnp.int32)
    msum2 = _sc_segsum(hbw2, srcc, dstc,
                       z2, z1, with_cnt=False)

    return _gate_call(h_split2, msum2, cnt2, raw1,
                      tmW1, tmb1, g1, beta1, W_out=W_out, b_out=b_out)
